# baseline (speedup 1.0000x reference)
import functools

import jax
import jax.numpy as jnp
from jax import lax
from jax.experimental import pallas as pl
from jax.experimental.pallas import tpu as pltpu
from jax.experimental.pallas import tpu_sc as plsc

DIM = 128
VOCAB = 225


def _sc_gather2(qw, kw, idx):
    nrows = idx.shape[0]
    info = plsc.get_sparse_core_info()
    nw = info.num_cores * info.num_subcores
    b_per_w = nrows // nw

    mesh = plsc.VectorSubcoreMesh(core_axis_name="c", subcore_axis_name="s")
    out_t = jax.ShapeDtypeStruct((nrows, DIM), jnp.float32)

    @functools.partial(
        pl.kernel,
        mesh=mesh,
        out_type=(out_t, out_t),
        scratch_types=[
            pltpu.VMEM((b_per_w,), jnp.int32),
            pltpu.VMEM((b_per_w, DIM), jnp.float32),
            pltpu.VMEM((b_per_w, DIM), jnp.float32),
            pltpu.SemaphoreType.DMA,
            pltpu.SemaphoreType.DMA,
            pltpu.SemaphoreType.DMA,
            pltpu.SemaphoreType.DMA,
        ],
    )
    def k(q_hbm, k_hbm, idx_hbm, oq, ok, idx_v, rq, rk, s0, s1, t0, t1):
        wid = lax.axis_index("s") * info.num_cores + lax.axis_index("c")
        base = wid * b_per_w
        pltpu.sync_copy(idx_hbm.at[pl.ds(base, b_per_w)], idx_v)
        cq = pltpu.async_copy(q_hbm.at[idx_v], rq, s0)
        ck = pltpu.async_copy(k_hbm.at[idx_v], rk, s1)
        cq.wait()
        wq = pltpu.async_copy(rq, oq.at[pl.ds(base, b_per_w)], t0)
        ck.wait()
        wk = pltpu.async_copy(rk, ok.at[pl.ds(base, b_per_w)], t1)
        wq.wait()
        wk.wait()

    return k(qw, kw, idx)


def _tc_onehot_gather(vw, idx3):
    nblocks, _, bs = idx3.shape
    nrows = nblocks * bs

    def body(idx_ref, tab_ref, out_ref):
        idxb = idx_ref[0, 0, :]
        onehot = (idxb[:, None] == lax.broadcasted_iota(
            jnp.int32, (bs, VOCAB), 1)).astype(jnp.float32)
        out_ref[...] = jnp.dot(onehot, tab_ref[...],
                               precision=lax.Precision.HIGHEST,
                               preferred_element_type=jnp.float32)

    return pl.pallas_call(
        body,
        grid=(nblocks,),
        in_specs=[
            pl.BlockSpec((1, 1, bs), lambda i: (i, 0, 0)),
            pl.BlockSpec((VOCAB, DIM), lambda i: (0, 0)),
        ],
        out_specs=pl.BlockSpec((bs, DIM), lambda i: (i, 0)),
        out_shape=jax.ShapeDtypeStruct((nrows, DIM), jnp.float32),
    )(idx3, vw)


def kernel(rel_q_weight, rel_k_weight, rel_v_weight, rel_index):
    idx = rel_index.reshape(-1).astype(jnp.int32)
    aq, ak = _sc_gather2(rel_q_weight, rel_k_weight, idx)
    av = _tc_onehot_gather(rel_v_weight, idx.reshape(16, 1, 256))
    shp = rel_index.shape + (DIM,)
    return aq.reshape(shp), ak.reshape(shp), av.reshape(shp)


# speedup vs baseline: 1.0345x; 1.0345x over previous
import functools

import jax
import jax.numpy as jnp
from jax import lax
from jax.experimental import pallas as pl
from jax.experimental.pallas import tpu as pltpu
from jax.experimental.pallas import tpu_sc as plsc

DIM = 128
VOCAB = 225
GRID = 64


def _sc_gather_q(qw, idx2d):
    nrows = GRID * GRID
    info = plsc.get_sparse_core_info()
    nw = info.num_cores * info.num_subcores
    b_per_w = nrows // nw
    rows_per_w = b_per_w // GRID

    mesh = plsc.VectorSubcoreMesh(core_axis_name="c", subcore_axis_name="s")

    @functools.partial(
        pl.kernel,
        mesh=mesh,
        out_type=jax.ShapeDtypeStruct((nrows, DIM), jnp.float32),
        scratch_types=[
            pltpu.VMEM((rows_per_w, GRID), jnp.int32),
            pltpu.VMEM((b_per_w, DIM), jnp.float32),
            pltpu.SemaphoreType.DMA,
            pltpu.SemaphoreType.DMA,
        ],
    )
    def k(q_hbm, idx_hbm, oq, idx_v, rq, s0, s1):
        wid = lax.axis_index("s") * info.num_cores + lax.axis_index("c")
        base = wid * b_per_w
        pltpu.sync_copy(idx_hbm.at[pl.ds(wid * rows_per_w, rows_per_w)], idx_v)
        c0 = pltpu.async_copy(q_hbm.at[idx_v.at[0]], rq.at[pl.ds(0, GRID)], s0)
        c1 = pltpu.async_copy(q_hbm.at[idx_v.at[1]], rq.at[pl.ds(GRID, GRID)], s1)
        c0.wait()
        c1.wait()
        pltpu.sync_copy(rq, oq.at[pl.ds(base, b_per_w)])

    return k(qw, idx2d)


def _tc_onehot_gather2(kw, vw, idx3):
    nblocks, _, bs = idx3.shape
    nrows = nblocks * bs

    def split3(tab):
        hi = tab.astype(jnp.bfloat16)
        r1 = tab - hi.astype(jnp.float32)
        mid = r1.astype(jnp.bfloat16)
        r2 = r1 - mid.astype(jnp.float32)
        return hi, mid, r2.astype(jnp.bfloat16)

    def body(idx_ref, ktab_ref, vtab_ref, ok_ref, ov_ref):
        idxb = idx_ref[0, 0, :]
        onehot = (idxb[:, None] == lax.broadcasted_iota(
            jnp.int32, (bs, VOCAB), 1)).astype(jnp.bfloat16)

        def sel(tab_ref):
            hi, mid, lo = split3(tab_ref[...])
            acc = jnp.dot(onehot, hi, preferred_element_type=jnp.float32)
            acc += jnp.dot(onehot, mid, preferred_element_type=jnp.float32)
            acc += jnp.dot(onehot, lo, preferred_element_type=jnp.float32)
            return acc

        ok_ref[...] = sel(ktab_ref)
        ov_ref[...] = sel(vtab_ref)

    out_t = jax.ShapeDtypeStruct((nrows, DIM), jnp.float32)
    return pl.pallas_call(
        body,
        grid=(nblocks,),
        in_specs=[
            pl.BlockSpec((1, 1, bs), lambda i: (i, 0, 0)),
            pl.BlockSpec((VOCAB, DIM), lambda i: (0, 0)),
            pl.BlockSpec((VOCAB, DIM), lambda i: (0, 0)),
        ],
        out_specs=[
            pl.BlockSpec((bs, DIM), lambda i: (i, 0)),
            pl.BlockSpec((bs, DIM), lambda i: (i, 0)),
        ],
        out_shape=(out_t, out_t),
    )(idx3, kw, vw)


def kernel(rel_q_weight, rel_k_weight, rel_v_weight, rel_index):
    idx2d = rel_index.astype(jnp.int32)
    aq = _sc_gather_q(rel_q_weight, idx2d)
    ak, av = _tc_onehot_gather2(rel_k_weight, rel_v_weight,
                                idx2d.reshape(16, 1, 256))
    shp = rel_index.shape + (DIM,)
    return aq.reshape(shp), ak.reshape(shp), av.reshape(shp)
